# baseline (speedup 1.0000x reference)
import functools

import jax
import jax.numpy as jnp
from jax import lax
from jax.experimental import pallas as pl
from jax.experimental.pallas import tpu as pltpu
from jax.experimental.pallas import tpu_sc as plsc

N_NODES = 10000
N_EDGES = 320000
D = 128

NW = 32
NLOC = 320
CE = 4000
NCHUNK = N_EDGES // CE
BATCH = 128

NEG_INF = float("-inf")


_RB = 1000


def _mm_body(x_ref, norm_ref, wa_ref, ba_ref, wb_ref, bb_ref,
             h_ref, ah_ref, bh_ref):
    h = x_ref[...] * norm_ref[...]
    h_ref[...] = h
    ah_ref[...] = jnp.dot(h, wa_ref[...],
                          preferred_element_type=jnp.float32) + ba_ref[...]
    bh_ref[...] = jnp.dot(h, wb_ref[...],
                          preferred_element_type=jnp.float32) + bb_ref[...]


def _tc_matmuls(x, norm, W_A, b_A, W_B, b_B):
    grid = (N_NODES // _RB,)
    return pl.pallas_call(
        _mm_body,
        grid=grid,
        in_specs=[
            pl.BlockSpec((_RB, D), lambda i: (i, 0)),
            pl.BlockSpec((_RB, 1), lambda i: (i, 0)),
            pl.BlockSpec((D, D), lambda i: (0, 0)),
            pl.BlockSpec((1, D), lambda i: (0, 0)),
            pl.BlockSpec((D, D), lambda i: (0, 0)),
            pl.BlockSpec((1, D), lambda i: (0, 0)),
        ],
        out_specs=[
            pl.BlockSpec((_RB, D), lambda i: (i, 0)),
            pl.BlockSpec((_RB, D), lambda i: (i, 0)),
            pl.BlockSpec((_RB, D), lambda i: (i, 0)),
        ],
        out_shape=[jax.ShapeDtypeStruct((N_NODES, D), jnp.float32)] * 3,
    )(x, norm, W_A, b_A.reshape(1, D), W_B, b_B.reshape(1, D))


def _sc_edge_body(ah_hbm, bh_hbm, src_hbm, dst_hbm, out_hbm,
                  acc, dstc, srcc, mdst, msrc, ahs, bhs, bhd,
                  sem_a, sem_b, sem_c):
    wid = lax.axis_index("s") * 2 + lax.axis_index("c")
    lo = wid * NLOC
    iota = lax.iota(jnp.int32, 16)
    lo_splat = jnp.full((16,), lo, jnp.int32)
    ninf = jnp.full((16,), NEG_INF, jnp.float32)

    def init_body(r, _):
        rsp = jnp.full((16,), r, jnp.int32)
        for j in range(8):
            plsc.store_scatter(acc, [rsp, iota + 16 * j], ninf)
        return 0
    lax.fori_loop(0, NLOC, init_body, 0)

    def chunk_body(ch, _):
        ebase = ch * CE
        pltpu.sync_copy(dst_hbm.at[pl.ds(ebase, CE)], dstc)
        pltpu.sync_copy(src_hbm.at[pl.ds(ebase, CE)], srcc)

        def scan_body(i, off_v):
            dv = plsc.load_gather(dstc, [iota + i * 16])
            sv = plsc.load_gather(srcc, [iota + i * 16])
            rel = dv - lo_splat
            m = (rel >= 0) & (rel < NLOC)
            cs = plsc.cumsum(m.astype(jnp.int32))
            pos = off_v + cs - 1
            plsc.store_scatter(mdst, [pos], dv, mask=m)
            plsc.store_scatter(msrc, [pos], sv, mask=m)
            return off_v + plsc.all_reduce_population_count(m)

        off_v = lax.fori_loop(0, CE // 16, scan_body,
                              jnp.zeros((16,), jnp.int32))
        cnt = jnp.max(off_v)

        for k in range(BATCH // 16):
            pidx = off_v + iota + 16 * k
            plsc.store_scatter(mdst, [pidx], lo_splat)
            plsc.store_scatter(msrc, [pidx], lo_splat)

        nb = (cnt + BATCH - 1) // BATCH

        def batch_body(b, _):
            base = b * BATCH
            cp1 = pltpu.async_copy(
                ah_hbm.at[msrc.at[pl.ds(base, BATCH)]], ahs, sem_a)
            cp2 = pltpu.async_copy(
                bh_hbm.at[msrc.at[pl.ds(base, BATCH)]], bhs, sem_b)
            cp3 = pltpu.async_copy(
                bh_hbm.at[mdst.at[pl.ds(base, BATCH)]], bhd, sem_c)
            cp1.wait()
            cp2.wait()
            cp3.wait()
            ew = jnp.minimum(BATCH, cnt - base)

            def edge_body(el, _):
                esp = jnp.full((16,), el, jnp.int32)
                gsp = esp + base
                relrow = plsc.load_gather(mdst, [gsp]) - lo_splat
                for j in range(8):
                    col = iota + 16 * j
                    av = plsc.load_gather(ahs, [esp, col])
                    bs = plsc.load_gather(bhs, [esp, col])
                    bd = plsc.load_gather(bhd, [esp, col])
                    msg = av / (1.0 + jnp.exp(-(bs + bd)))
                    cur = plsc.load_gather(acc, [relrow, col])
                    plsc.store_scatter(acc, [relrow, col],
                                       jnp.maximum(cur, msg))
                return 0

            lax.fori_loop(0, ew, edge_body, 0)
            return 0

        lax.fori_loop(0, nb, batch_body, 0)
        return 0

    lax.fori_loop(0, NCHUNK, chunk_body, 0)

    last_rows = N_NODES - (NW - 1) * NLOC

    @pl.when(wid < NW - 1)
    def _():
        pltpu.sync_copy(acc.at[pl.ds(0, NLOC)], out_hbm.at[pl.ds(lo, NLOC)])

    @pl.when(wid == NW - 1)
    def _():
        pltpu.sync_copy(acc.at[pl.ds(0, last_rows)],
                        out_hbm.at[pl.ds((NW - 1) * NLOC, last_rows)])


def _sc_edge(ah, bh, src, dst):
    mesh = plsc.VectorSubcoreMesh(core_axis_name="c", subcore_axis_name="s")
    return pl.kernel(
        _sc_edge_body,
        out_type=jax.ShapeDtypeStruct((N_NODES, D), jnp.float32),
        mesh=mesh,
        compiler_params=pltpu.CompilerParams(needs_layout_passes=False),
        scratch_types=[
            pltpu.VMEM((NLOC, D), jnp.float32),
            pltpu.VMEM((CE,), jnp.int32),
            pltpu.VMEM((CE,), jnp.int32),
            pltpu.VMEM((CE + BATCH,), jnp.int32),
            pltpu.VMEM((CE + BATCH,), jnp.int32),
            pltpu.VMEM((BATCH, D), jnp.float32),
            pltpu.VMEM((BATCH, D), jnp.float32),
            pltpu.VMEM((BATCH, D), jnp.float32),
            pltpu.SemaphoreType.DMA,
            pltpu.SemaphoreType.DMA,
            pltpu.SemaphoreType.DMA,
        ],
    )(ah, bh, src, dst)


def _norm_body(h_ref, c_ref, norm_ref, out_ref):
    h = h_ref[...]
    c = c_ref[...]
    c = jnp.where(jnp.isfinite(c), c, 0.0)
    n2 = jnp.sum(h * h, axis=1, keepdims=True) + \
        jnp.sum(c * c, axis=1, keepdims=True)
    denom = jnp.maximum(jnp.sqrt(n2), 1e-12)
    scale = norm_ref[...] / denom
    out_ref[:, :D] = h * scale
    out_ref[:, D:] = c * scale


def _tc_normalize(h, c, norm):
    grid = (N_NODES // _RB,)
    return pl.pallas_call(
        _norm_body,
        grid=grid,
        in_specs=[
            pl.BlockSpec((_RB, D), lambda i: (i, 0)),
            pl.BlockSpec((_RB, D), lambda i: (i, 0)),
            pl.BlockSpec((_RB, 1), lambda i: (i, 0)),
        ],
        out_specs=pl.BlockSpec((_RB, 2 * D), lambda i: (i, 0)),
        out_shape=jax.ShapeDtypeStruct((N_NODES, 2 * D), jnp.float32),
    )(h, c, norm)


def kernel(x, edge_index, norm, W_A, b_A, W_B, b_B):
    src = edge_index[0].astype(jnp.int32)
    dst = edge_index[1].astype(jnp.int32)
    h, ah, bh = _tc_matmuls(x, norm, W_A, b_A, W_B, b_B)
    c = _sc_edge(ah, bh, src, dst)
    return _tc_normalize(h, c, norm)


# speedup vs baseline: 1.8848x; 1.8848x over previous
import jax
import jax.numpy as jnp
from jax import lax
from jax.experimental import pallas as pl
from jax.experimental.pallas import tpu as pltpu
from jax.experimental.pallas import tpu_sc as plsc

N_NODES = 10000
N_EDGES = 320000
D = 128

NW = 32
NLOC = 320
CE = 1600
NCHUNK = N_EDGES // CE
BATCH = 64
UNROLL = 4

NEG_INF = float("-inf")


_RB = 1000


def _mm_body(x_ref, norm_ref, wa_ref, ba_ref, wb_ref, bb_ref,
             h_ref, ab_ref, bh_ref):
    h = x_ref[...] * norm_ref[...]
    h_ref[...] = h
    ah = jnp.dot(h, wa_ref[...], preferred_element_type=jnp.float32) \
        + ba_ref[...]
    bh = jnp.dot(h, wb_ref[...], preferred_element_type=jnp.float32) \
        + bb_ref[...]
    ab_ref[:, :D] = ah
    ab_ref[:, D:] = bh
    bh_ref[...] = bh


def _tc_matmuls(x, norm, W_A, b_A, W_B, b_B):
    grid = (N_NODES // _RB,)
    return pl.pallas_call(
        _mm_body,
        grid=grid,
        in_specs=[
            pl.BlockSpec((_RB, D), lambda i: (i, 0)),
            pl.BlockSpec((_RB, 1), lambda i: (i, 0)),
            pl.BlockSpec((D, D), lambda i: (0, 0)),
            pl.BlockSpec((1, D), lambda i: (0, 0)),
            pl.BlockSpec((D, D), lambda i: (0, 0)),
            pl.BlockSpec((1, D), lambda i: (0, 0)),
        ],
        out_specs=[
            pl.BlockSpec((_RB, D), lambda i: (i, 0)),
            pl.BlockSpec((_RB, 2 * D), lambda i: (i, 0)),
            pl.BlockSpec((_RB, D), lambda i: (i, 0)),
        ],
        out_shape=[
            jax.ShapeDtypeStruct((N_NODES, D), jnp.float32),
            jax.ShapeDtypeStruct((N_NODES, 2 * D), jnp.float32),
            jax.ShapeDtypeStruct((N_NODES, D), jnp.float32),
        ],
    )(x, norm, W_A, b_A.reshape(1, D), W_B, b_B.reshape(1, D))


def _sc_edge_body(ab_hbm, bh_hbm, src_hbm, dst_hbm, out_hbm,
                  acc0, acc1, acc2, acc3, acc4, acc5, acc6, acc7,
                  bhloc, dstc, srcc, mdst, msrc, absbuf, sem_a):
    accs = (acc0, acc1, acc2, acc3, acc4, acc5, acc6, acc7)
    wid = lax.axis_index("s") * 2 + lax.axis_index("c")
    lo = wid * NLOC
    iota = lax.iota(jnp.int32, 16)
    cols = [iota + 16 * j for j in range(8)]
    lo_splat = jnp.full((16,), lo, jnp.int32)
    ninf = jnp.full((16,), NEG_INF, jnp.float32)
    last_rows = N_NODES - (NW - 1) * NLOC

    @pl.when(wid < NW - 1)
    def _():
        pltpu.sync_copy(bh_hbm.at[pl.ds(lo, NLOC)], bhloc)

    @pl.when(wid == NW - 1)
    def _():
        pltpu.sync_copy(bh_hbm.at[pl.ds((NW - 1) * NLOC, last_rows)],
                        bhloc.at[pl.ds(0, last_rows)])

    def init_body(r, _):
        idx = iota + r * 16
        for j in range(8):
            plsc.store_scatter(accs[j], [idx], ninf)
        return 0
    lax.fori_loop(0, NLOC, init_body, 0)

    def chunk_body(ch, _):
        ebase = ch * CE
        pltpu.sync_copy(dst_hbm.at[pl.ds(ebase, CE)], dstc)
        pltpu.sync_copy(src_hbm.at[pl.ds(ebase, CE)], srcc)

        def scan_body(i, off_v):
            dv = plsc.load_gather(dstc, [iota + i * 16])
            sv = plsc.load_gather(srcc, [iota + i * 16])
            rel = dv - lo_splat
            m = (rel >= 0) & (rel < NLOC)
            cs = plsc.cumsum(m.astype(jnp.int32))
            pos = off_v + cs - 1
            plsc.store_scatter(mdst, [pos], dv, mask=m)
            plsc.store_scatter(msrc, [pos], sv, mask=m)
            return off_v + plsc.all_reduce_population_count(m)

        off_v = lax.fori_loop(0, CE // 16, scan_body,
                              jnp.zeros((16,), jnp.int32))
        cnt = jnp.max(off_v)
        cnt_sp = jnp.full((16,), cnt, jnp.int32)

        for k in range(BATCH // 16):
            pidx = off_v + iota + 16 * k
            plsc.store_scatter(mdst, [pidx], lo_splat)
            plsc.store_scatter(msrc, [pidx], lo_splat)

        nb = (cnt + BATCH - 1) // BATCH

        def batch_body(b, _):
            base = b * BATCH
            pltpu.async_copy(
                ab_hbm.at[msrc.at[pl.ds(base, BATCH)]], absbuf, sem_a).wait()

            def quad_body(it, _):
                edges = []
                for u in range(UNROLL):
                    el = it * UNROLL + u
                    esp = jnp.full((16,), base + el, jnp.int32)
                    lsp = jnp.full((16,), el, jnp.int32)
                    rel = plsc.load_gather(mdst, [esp]) - lo_splat
                    aidx = (rel << 4) + iota
                    valid = esp < cnt_sp
                    edges.append((lsp, rel, aidx, valid))
                J = range(8)
                bb = [[plsc.load_gather(absbuf, [e[0], cols[j] + 128])
                       for j in J] for e in edges]
                bd = [[plsc.load_gather(bhloc, [e[1], cols[j]])
                       for j in J] for e in edges]
                t = [[bb[u][j] + bd[u][j] for j in J] for u in range(UNROLL)]
                ex = [[jnp.exp(-t[u][j]) for j in J] for u in range(UNROLL)]
                dn = [[1.0 + ex[u][j] for j in J] for u in range(UNROLL)]
                iv = [[1.0 / dn[u][j] for j in J] for u in range(UNROLL)]
                av = [[plsc.load_gather(absbuf, [e[0], cols[j]])
                       for j in J] for e in edges]
                for u, e in enumerate(edges):
                    for j in J:
                        msg = av[u][j] * iv[u][j]
                        cur = plsc.load_gather(accs[j], [e[2]])
                        plsc.store_scatter(accs[j], [e[2]],
                                           jnp.maximum(cur, msg), mask=e[3])
                return 0

            lax.fori_loop(0, BATCH // UNROLL, quad_body, 0)
            return 0

        lax.fori_loop(0, nb, batch_body, 0)
        return 0

    lax.fori_loop(0, NCHUNK, chunk_body, 0)

    def regroup_body(r, _):
        rsp = jnp.full((16,), r, jnp.int32)
        idx = iota + r * 16
        for j in range(8):
            v = plsc.load_gather(accs[j], [idx])
            plsc.store_scatter(bhloc, [rsp, cols[j]], v)
        return 0
    lax.fori_loop(0, NLOC, regroup_body, 0)

    @pl.when(wid < NW - 1)
    def _():
        pltpu.sync_copy(bhloc, out_hbm.at[pl.ds(lo, NLOC)])

    @pl.when(wid == NW - 1)
    def _():
        pltpu.sync_copy(bhloc.at[pl.ds(0, last_rows)],
                        out_hbm.at[pl.ds((NW - 1) * NLOC, last_rows)])


def _sc_edge(ab, bh, src, dst):
    mesh = plsc.VectorSubcoreMesh(core_axis_name="c", subcore_axis_name="s")
    return pl.kernel(
        _sc_edge_body,
        out_type=jax.ShapeDtypeStruct((N_NODES, D), jnp.float32),
        mesh=mesh,
        compiler_params=pltpu.CompilerParams(needs_layout_passes=False),
        scratch_types=[
            *[pltpu.VMEM((NLOC * 16,), jnp.float32) for _ in range(8)],
            pltpu.VMEM((NLOC, D), jnp.float32),
            pltpu.VMEM((CE,), jnp.int32),
            pltpu.VMEM((CE,), jnp.int32),
            pltpu.VMEM((CE + BATCH,), jnp.int32),
            pltpu.VMEM((CE + BATCH,), jnp.int32),
            pltpu.VMEM((BATCH, 2 * D), jnp.float32),
            pltpu.SemaphoreType.DMA,
        ],
    )(ab, bh, src, dst)


def _norm_body(h_ref, c_ref, norm_ref, out_ref):
    h = h_ref[...]
    c = c_ref[...]
    c = jnp.where(jnp.isfinite(c), c, 0.0)
    n2 = jnp.sum(h * h, axis=1, keepdims=True) + \
        jnp.sum(c * c, axis=1, keepdims=True)
    denom = jnp.maximum(jnp.sqrt(n2), 1e-12)
    scale = norm_ref[...] / denom
    out_ref[:, :D] = h * scale
    out_ref[:, D:] = c * scale


def _tc_normalize(h, c, norm):
    grid = (N_NODES // _RB,)
    return pl.pallas_call(
        _norm_body,
        grid=grid,
        in_specs=[
            pl.BlockSpec((_RB, D), lambda i: (i, 0)),
            pl.BlockSpec((_RB, D), lambda i: (i, 0)),
            pl.BlockSpec((_RB, 1), lambda i: (i, 0)),
        ],
        out_specs=pl.BlockSpec((_RB, 2 * D), lambda i: (i, 0)),
        out_shape=jax.ShapeDtypeStruct((N_NODES, 2 * D), jnp.float32),
    )(h, c, norm)


def kernel(x, edge_index, norm, W_A, b_A, W_B, b_B):
    src = edge_index[0].astype(jnp.int32)
    dst = edge_index[1].astype(jnp.int32)
    h, ab, bh = _tc_matmuls(x, norm, W_A, b_A, W_B, b_B)
    c = _sc_edge(ab, bh, src, dst)
    return _tc_normalize(h, c, norm)


# speedup vs baseline: 2.8630x; 1.5190x over previous
import jax
import jax.numpy as jnp
from jax import lax
from jax.experimental import pallas as pl
from jax.experimental.pallas import tpu as pltpu
from jax.experimental.pallas import tpu_sc as plsc

N_NODES = 10000
N_EDGES = 320000
D = 128

NW = 32
NLOC = 320
CE = 1600
NCHUNK = N_EDGES // CE
BATCH = 64
UNROLL = 4

NEG_INF = float("-inf")


_RB = 1000


def _mm_body(x_ref, norm_ref, wa_ref, ba_ref, wb_ref, bb_ref,
             h_ref, ab_ref, bh_ref):
    h = x_ref[...] * norm_ref[...]
    h_ref[...] = h
    ah = jnp.dot(h, wa_ref[...], preferred_element_type=jnp.float32) \
        + ba_ref[...]
    bh = jnp.dot(h, wb_ref[...], preferred_element_type=jnp.float32) \
        + bb_ref[...]
    ab_ref[:, :D] = ah
    ab_ref[:, D:] = bh
    bh_ref[...] = bh


def _tc_matmuls(x, norm, W_A, b_A, W_B, b_B):
    grid = (N_NODES // _RB,)
    return pl.pallas_call(
        _mm_body,
        grid=grid,
        in_specs=[
            pl.BlockSpec((_RB, D), lambda i: (i, 0)),
            pl.BlockSpec((_RB, 1), lambda i: (i, 0)),
            pl.BlockSpec((D, D), lambda i: (0, 0)),
            pl.BlockSpec((1, D), lambda i: (0, 0)),
            pl.BlockSpec((D, D), lambda i: (0, 0)),
            pl.BlockSpec((1, D), lambda i: (0, 0)),
        ],
        out_specs=[
            pl.BlockSpec((_RB, D), lambda i: (i, 0)),
            pl.BlockSpec((_RB, 2 * D), lambda i: (i, 0)),
            pl.BlockSpec((_RB, D), lambda i: (i, 0)),
        ],
        out_shape=[
            jax.ShapeDtypeStruct((N_NODES, D), jnp.float32),
            jax.ShapeDtypeStruct((N_NODES, 2 * D), jnp.float32),
            jax.ShapeDtypeStruct((N_NODES, D), jnp.float32),
        ],
    )(x, norm, W_A, b_A.reshape(1, D), W_B, b_B.reshape(1, D))


def _sc_edge_body(ab_hbm, bh_hbm, src_hbm, dst_hbm, out_hbm,
                  acc0, acc1, acc2, acc3, acc4, acc5, acc6, acc7,
                  bhloc, dstc0, srcc0, dstc1, srcc1,
                  mdst0, msrc0, mdst1, msrc1, absbuf0, absbuf1,
                  sem_stage, sem_ab0, sem_ab1, sem_slow):
    accs = (acc0, acc1, acc2, acc3, acc4, acc5, acc6, acc7)
    wid = lax.axis_index("s") * 2 + lax.axis_index("c")
    lo = wid * NLOC
    iota = lax.iota(jnp.int32, 16)
    cols = [iota + 16 * j for j in range(8)]
    lo_splat = jnp.full((16,), lo, jnp.int32)
    ninf = jnp.full((16,), NEG_INF, jnp.float32)
    last_rows = N_NODES - (NW - 1) * NLOC

    @pl.when(wid < NW - 1)
    def _():
        pltpu.sync_copy(bh_hbm.at[pl.ds(lo, NLOC)], bhloc)

    @pl.when(wid == NW - 1)
    def _():
        pltpu.sync_copy(bh_hbm.at[pl.ds((NW - 1) * NLOC, last_rows)],
                        bhloc.at[pl.ds(0, last_rows)])

    def init_body(r, _):
        idx = iota + r * 16
        for j in range(8):
            plsc.store_scatter(accs[j], [idx], ninf)
        return 0
    lax.fori_loop(0, NLOC, init_body, 0)

    _gdn = lax.GatherDimensionNumbers(
        offset_dims=(), collapsed_slice_dims=(0,), start_index_map=(0,))

    def take16(x, idx):
        return lax.gather(x, idx[:, None], _gdn, (1,),
                          mode=lax.GatherScatterMode.PROMISE_IN_BOUNDS)

    shifts = []
    for k in (1, 2, 4, 8):
        shifts.append((jnp.maximum(iota - k, 0), iota >= k))

    def prefix16(x):
        for sidx, keep in shifts:
            x = x + jnp.where(keep, take16(x, sidx), 0)
        return x

    def scan_chunk(ch, dstcr, srccr, mdst_r, msrc_r):
        def scan_body(i, off_v):
            dv = plsc.load_gather(dstcr, [iota + i * 16])
            sv = plsc.load_gather(srccr, [iota + i * 16])
            rel = dv - lo_splat
            m = (rel >= 0) & (rel < NLOC)
            cs = prefix16(m.astype(jnp.int32))
            pos = off_v + cs - 1
            plsc.store_scatter(mdst_r, [pos], dv, mask=m)
            plsc.store_scatter(msrc_r, [pos], sv, mask=m)
            return off_v + take16(cs, jnp.full((16,), 15, jnp.int32))

        off_v = lax.fori_loop(0, CE // 16, scan_body,
                              jnp.zeros((16,), jnp.int32))
        for k in range(BATCH // 16):
            pidx = off_v + iota + 16 * k
            plsc.store_scatter(mdst_r, [pidx], lo_splat)
            plsc.store_scatter(msrc_r, [pidx], lo_splat)
        return jnp.max(off_v)

    def stage_issue(ch, dstcr, srccr):
        ebase = ch * CE
        pltpu.async_copy(dst_hbm.at[pl.ds(ebase, CE)], dstcr, sem_stage)
        pltpu.async_copy(src_hbm.at[pl.ds(ebase, CE)], srccr, sem_stage)

    def stage_wait(dstcr, srccr):
        pltpu.make_async_copy(dst_hbm.at[pl.ds(0, CE)], dstcr,
                              sem_stage).wait()
        pltpu.make_async_copy(src_hbm.at[pl.ds(0, CE)], srccr,
                              sem_stage).wait()

    def process(absref, mdst_r, msrc_r, cnt):
        cnt_sp = jnp.full((16,), cnt, jnp.int32)
        nb = (cnt + BATCH - 1) // BATCH

        def batch_body(b, _):
            base = b * BATCH

            @pl.when(b > 0)
            def _():
                pltpu.async_copy(
                    ab_hbm.at[msrc_r.at[pl.ds(base, BATCH)]], absref,
                    sem_slow).wait()

            def quad_body(it, _):
                edges = []
                for u in range(UNROLL):
                    el = it * UNROLL + u
                    esp = jnp.full((16,), base + el, jnp.int32)
                    lsp = jnp.full((16,), el, jnp.int32)
                    rel = plsc.load_gather(mdst_r, [esp]) - lo_splat
                    aidx = (rel << 4) + iota
                    valid = esp < cnt_sp
                    edges.append((lsp, rel, aidx, valid))
                J = range(8)
                bb = [[plsc.load_gather(absref, [e[0], cols[j] + 128])
                       for j in J] for e in edges]
                bd = [[plsc.load_gather(bhloc, [e[1], cols[j]])
                       for j in J] for e in edges]
                t = [[bb[u][j] + bd[u][j] for j in J] for u in range(UNROLL)]
                ex = [[jnp.exp(-t[u][j]) for j in J] for u in range(UNROLL)]
                dn = [[1.0 + ex[u][j] for j in J] for u in range(UNROLL)]
                iv = [[1.0 / dn[u][j] for j in J] for u in range(UNROLL)]
                av = [[plsc.load_gather(absref, [e[0], cols[j]])
                       for j in J] for e in edges]
                for u, e in enumerate(edges):
                    for j in J:
                        msg = av[u][j] * iv[u][j]
                        cur = plsc.load_gather(accs[j], [e[2]])
                        plsc.store_scatter(accs[j], [e[2]],
                                           jnp.maximum(cur, msg), mask=e[3])
                return 0

            nq = jnp.minimum((cnt - base + UNROLL - 1) >> 2, BATCH // UNROLL)
            lax.fori_loop(0, nq, quad_body, 0)
            return 0

        lax.fori_loop(0, nb, batch_body, 0)

    stage_issue(0, dstc0, srcc0)

    def pair_body(k, cnt_o):
        ch = 2 * k
        stage_wait(dstc0, srcc0)
        stage_issue(ch + 1, dstc1, srcc1)
        cnt_e = scan_chunk(ch, dstc0, srcc0, mdst0, msrc0)
        d_ab0 = pltpu.async_copy(
            ab_hbm.at[msrc0.at[pl.ds(0, BATCH)]], absbuf0, sem_ab0)

        @pl.when(k > 0)
        def _():
            pltpu.make_async_copy(
                ab_hbm.at[msrc1.at[pl.ds(0, BATCH)]], absbuf1, sem_ab1).wait()
            process(absbuf1, mdst1, msrc1, cnt_o)

        stage_wait(dstc1, srcc1)

        @pl.when(k < NCHUNK // 2 - 1)
        def _():
            stage_issue(ch + 2, dstc0, srcc0)
        cnt_o_new = scan_chunk(ch + 1, dstc1, srcc1, mdst1, msrc1)
        pltpu.async_copy(
            ab_hbm.at[msrc1.at[pl.ds(0, BATCH)]], absbuf1, sem_ab1)

        d_ab0.wait()
        process(absbuf0, mdst0, msrc0, cnt_e)
        return cnt_o_new

    cnt_o = lax.fori_loop(0, NCHUNK // 2, pair_body, jnp.int32(0))
    pltpu.make_async_copy(
        ab_hbm.at[msrc1.at[pl.ds(0, BATCH)]], absbuf1, sem_ab1).wait()
    process(absbuf1, mdst1, msrc1, cnt_o)

    def regroup_body(r, _):
        rsp = jnp.full((16,), r, jnp.int32)
        idx = iota + r * 16
        for j in range(8):
            v = plsc.load_gather(accs[j], [idx])
            plsc.store_scatter(bhloc, [rsp, cols[j]], v)
        return 0
    lax.fori_loop(0, NLOC, regroup_body, 0)

    @pl.when(wid < NW - 1)
    def _():
        pltpu.sync_copy(bhloc, out_hbm.at[pl.ds(lo, NLOC)])

    @pl.when(wid == NW - 1)
    def _():
        pltpu.sync_copy(bhloc.at[pl.ds(0, last_rows)],
                        out_hbm.at[pl.ds((NW - 1) * NLOC, last_rows)])


def _sc_edge(ab, bh, src, dst):
    mesh = plsc.VectorSubcoreMesh(core_axis_name="c", subcore_axis_name="s")
    return pl.kernel(
        _sc_edge_body,
        out_type=jax.ShapeDtypeStruct((N_NODES, D), jnp.float32),
        mesh=mesh,
        compiler_params=pltpu.CompilerParams(needs_layout_passes=False),
        scratch_types=[
            *[pltpu.VMEM((NLOC * 16,), jnp.float32) for _ in range(8)],
            pltpu.VMEM((NLOC, D), jnp.float32),
            pltpu.VMEM((CE,), jnp.int32),
            pltpu.VMEM((CE,), jnp.int32),
            pltpu.VMEM((CE,), jnp.int32),
            pltpu.VMEM((CE,), jnp.int32),
            pltpu.VMEM((CE + BATCH,), jnp.int32),
            pltpu.VMEM((CE + BATCH,), jnp.int32),
            pltpu.VMEM((CE + BATCH,), jnp.int32),
            pltpu.VMEM((CE + BATCH,), jnp.int32),
            pltpu.VMEM((BATCH, 2 * D), jnp.float32),
            pltpu.VMEM((BATCH, 2 * D), jnp.float32),
            pltpu.SemaphoreType.DMA,
            pltpu.SemaphoreType.DMA,
            pltpu.SemaphoreType.DMA,
            pltpu.SemaphoreType.DMA,
        ],
    )(ab, bh, src, dst)


def _norm_body(h_ref, c_ref, norm_ref, out_ref):
    h = h_ref[...]
    c = c_ref[...]
    c = jnp.where(jnp.isfinite(c), c, 0.0)
    n2 = jnp.sum(h * h, axis=1, keepdims=True) + \
        jnp.sum(c * c, axis=1, keepdims=True)
    denom = jnp.maximum(jnp.sqrt(n2), 1e-12)
    scale = norm_ref[...] / denom
    out_ref[:, :D] = h * scale
    out_ref[:, D:] = c * scale


def _tc_normalize(h, c, norm):
    grid = (N_NODES // _RB,)
    return pl.pallas_call(
        _norm_body,
        grid=grid,
        in_specs=[
            pl.BlockSpec((_RB, D), lambda i: (i, 0)),
            pl.BlockSpec((_RB, D), lambda i: (i, 0)),
            pl.BlockSpec((_RB, 1), lambda i: (i, 0)),
        ],
        out_specs=pl.BlockSpec((_RB, 2 * D), lambda i: (i, 0)),
        out_shape=jax.ShapeDtypeStruct((N_NODES, 2 * D), jnp.float32),
    )(h, c, norm)


def kernel(x, edge_index, norm, W_A, b_A, W_B, b_B):
    src = edge_index[0].astype(jnp.int32)
    dst = edge_index[1].astype(jnp.int32)
    h, ab, bh = _tc_matmuls(x, norm, W_A, b_A, W_B, b_B)
    c = _sc_edge(ab, bh, src, dst)
    return _tc_normalize(h, c, norm)


# speedup vs baseline: 3.5683x; 1.2463x over previous
import jax
import jax.numpy as jnp
from jax import lax
from jax.experimental import pallas as pl
from jax.experimental.pallas import tpu as pltpu
from jax.experimental.pallas import tpu_sc as plsc

N_NODES = 10000
N_EDGES = 320000
D = 128

NW = 32
NLOC = 320
CE = 1600
NCHUNK = N_EDGES // CE
BATCH = 64
UNROLL = 2

NEG_INF = float("-inf")


_RB = 1000


def _mm_body(x_ref, norm_ref, wa_ref, ba_ref, wb_ref, bb_ref,
             h_ref, ab_ref, bh_ref):
    h = x_ref[...] * norm_ref[...]
    h_ref[...] = h
    ah = jnp.dot(h, wa_ref[...], preferred_element_type=jnp.float32) \
        + ba_ref[...]
    bh = jnp.dot(h, wb_ref[...], preferred_element_type=jnp.float32) \
        + bb_ref[...]
    ab_ref[:, :D] = ah
    ab_ref[:, D:] = bh
    bh_ref[...] = bh


def _tc_matmuls(x, norm, W_A, b_A, W_B, b_B):
    grid = (N_NODES // _RB,)
    return pl.pallas_call(
        _mm_body,
        grid=grid,
        in_specs=[
            pl.BlockSpec((_RB, D), lambda i: (i, 0)),
            pl.BlockSpec((_RB, 1), lambda i: (i, 0)),
            pl.BlockSpec((D, D), lambda i: (0, 0)),
            pl.BlockSpec((1, D), lambda i: (0, 0)),
            pl.BlockSpec((D, D), lambda i: (0, 0)),
            pl.BlockSpec((1, D), lambda i: (0, 0)),
        ],
        out_specs=[
            pl.BlockSpec((_RB, D), lambda i: (i, 0)),
            pl.BlockSpec((_RB, 2 * D), lambda i: (i, 0)),
            pl.BlockSpec((_RB, D), lambda i: (i, 0)),
        ],
        out_shape=[
            jax.ShapeDtypeStruct((N_NODES, D), jnp.float32),
            jax.ShapeDtypeStruct((N_NODES, 2 * D), jnp.float32),
            jax.ShapeDtypeStruct((N_NODES, D), jnp.float32),
        ],
    )(x, norm, W_A, b_A.reshape(1, D), W_B, b_B.reshape(1, D))


def _sc_edge_body(ab_hbm, bh_hbm, src_hbm, dst_hbm, out_hbm,
                  acc0, acc1, acc2, acc3, acc4, acc5, acc6, acc7,
                  bhloc, dstc0, srcc0, dstc1, srcc1,
                  mdst0, msrc0, mdst1, msrc1, absbuf0, absbuf1,
                  sem_stage, sem_ab0, sem_ab1, sem_slow):
    accs = (acc0, acc1, acc2, acc3, acc4, acc5, acc6, acc7)
    wid = lax.axis_index("s") * 2 + lax.axis_index("c")
    lo = wid * NLOC
    iota = lax.iota(jnp.int32, 16)
    cols = [iota + 16 * j for j in range(8)]
    lo_splat = jnp.full((16,), lo, jnp.int32)
    ninf = jnp.full((16,), NEG_INF, jnp.float32)
    last_rows = N_NODES - (NW - 1) * NLOC

    @pl.when(wid < NW - 1)
    def _():
        pltpu.sync_copy(bh_hbm.at[pl.ds(lo, NLOC)], bhloc)

    @pl.when(wid == NW - 1)
    def _():
        pltpu.sync_copy(bh_hbm.at[pl.ds((NW - 1) * NLOC, last_rows)],
                        bhloc.at[pl.ds(0, last_rows)])

    def init_body(r, _):
        idx = iota + r * 16
        for j in range(8):
            plsc.store_scatter(accs[j], [idx], ninf)
        return 0
    lax.fori_loop(0, NLOC, init_body, 0)

    _gdn = lax.GatherDimensionNumbers(
        offset_dims=(), collapsed_slice_dims=(0,), start_index_map=(0,))

    def take16(x, idx):
        return lax.gather(x, idx[:, None], _gdn, (1,),
                          mode=lax.GatherScatterMode.PROMISE_IN_BOUNDS)

    shifts = []
    for k in (1, 2, 4, 8):
        shifts.append((jnp.maximum(iota - k, 0), iota >= k))

    def prefix16(x):
        for sidx, keep in shifts:
            x = x + jnp.where(keep, take16(x, sidx), 0)
        return x

    def scan_chunk(ch, dstcr, srccr, mdst_r, msrc_r):
        def scan_body(i, off_v):
            dv = plsc.load_gather(dstcr, [iota + i * 16])
            sv = plsc.load_gather(srccr, [iota + i * 16])
            rel = dv - lo_splat
            m = (rel >= 0) & (rel < NLOC)
            cs = prefix16(m.astype(jnp.int32))
            pos = off_v + cs - 1
            plsc.store_scatter(mdst_r, [pos], dv, mask=m)
            plsc.store_scatter(msrc_r, [pos], sv, mask=m)
            return off_v + take16(cs, jnp.full((16,), 15, jnp.int32))

        off_v = lax.fori_loop(0, CE // 16, scan_body,
                              jnp.zeros((16,), jnp.int32))
        for k in range(BATCH // 16):
            pidx = off_v + iota + 16 * k
            plsc.store_scatter(mdst_r, [pidx], lo_splat)
            plsc.store_scatter(msrc_r, [pidx], lo_splat)
        return jnp.max(off_v)

    def stage_issue(ch, dstcr, srccr):
        ebase = ch * CE
        pltpu.async_copy(dst_hbm.at[pl.ds(ebase, CE)], dstcr, sem_stage)
        pltpu.async_copy(src_hbm.at[pl.ds(ebase, CE)], srccr, sem_stage)

    def stage_wait(dstcr, srccr):
        pltpu.make_async_copy(dst_hbm.at[pl.ds(0, CE)], dstcr,
                              sem_stage).wait()
        pltpu.make_async_copy(src_hbm.at[pl.ds(0, CE)], srccr,
                              sem_stage).wait()

    def process(absref, mdst_r, msrc_r, cnt):
        cnt_sp = jnp.full((16,), cnt, jnp.int32)
        nb = (cnt + BATCH - 1) // BATCH

        def batch_body(b, _):
            base = b * BATCH

            @pl.when(b > 0)
            def _():
                pltpu.async_copy(
                    ab_hbm.at[msrc_r.at[pl.ds(base, BATCH)]], absref,
                    sem_slow).wait()

            def quad_body(it, _):
                edges = []
                for u in range(UNROLL):
                    el = it * UNROLL + u
                    esp = jnp.full((16,), base + el, jnp.int32)
                    lsp = jnp.full((16,), el, jnp.int32)
                    rel = plsc.load_gather(mdst_r, [esp]) - lo_splat
                    aidx = (rel << 4) + iota
                    valid = esp < cnt_sp
                    edges.append((lsp, rel, aidx, valid))
                J = range(8)
                bb = [[plsc.load_gather(absref, [e[0], cols[j] + 128])
                       for j in J] for e in edges]
                bd = [[plsc.load_gather(bhloc, [e[1], cols[j]])
                       for j in J] for e in edges]
                t = [[bb[u][j] + bd[u][j] for j in J] for u in range(UNROLL)]
                ex = [[jnp.exp(-t[u][j]) for j in J] for u in range(UNROLL)]
                dn = [[1.0 + ex[u][j] for j in J] for u in range(UNROLL)]
                iv = [[1.0 / dn[u][j] for j in J] for u in range(UNROLL)]
                av = [[plsc.load_gather(absref, [e[0], cols[j]])
                       for j in J] for e in edges]
                for u, e in enumerate(edges):
                    for j in J:
                        msg = av[u][j] * iv[u][j]
                        cur = plsc.load_gather(accs[j], [e[2]])
                        plsc.store_scatter(accs[j], [e[2]],
                                           jnp.maximum(cur, msg), mask=e[3])
                return 0

            nq = jnp.minimum((cnt - base + UNROLL - 1) >> 1, BATCH // UNROLL)
            lax.fori_loop(0, nq, quad_body, 0)
            return 0

        lax.fori_loop(0, nb, batch_body, 0)

    stage_issue(0, dstc0, srcc0)

    def pair_body(k, cnt_o):
        ch = 2 * k
        stage_wait(dstc0, srcc0)
        stage_issue(ch + 1, dstc1, srcc1)
        cnt_e = scan_chunk(ch, dstc0, srcc0, mdst0, msrc0)
        d_ab0 = pltpu.async_copy(
            ab_hbm.at[msrc0.at[pl.ds(0, BATCH)]], absbuf0, sem_ab0)

        @pl.when(k > 0)
        def _():
            pltpu.make_async_copy(
                ab_hbm.at[msrc1.at[pl.ds(0, BATCH)]], absbuf1, sem_ab1).wait()
            process(absbuf1, mdst1, msrc1, cnt_o)

        stage_wait(dstc1, srcc1)

        @pl.when(k < NCHUNK // 2 - 1)
        def _():
            stage_issue(ch + 2, dstc0, srcc0)
        cnt_o_new = scan_chunk(ch + 1, dstc1, srcc1, mdst1, msrc1)
        pltpu.async_copy(
            ab_hbm.at[msrc1.at[pl.ds(0, BATCH)]], absbuf1, sem_ab1)

        d_ab0.wait()
        process(absbuf0, mdst0, msrc0, cnt_e)
        return cnt_o_new

    cnt_o = lax.fori_loop(0, NCHUNK // 2, pair_body, jnp.int32(0))
    pltpu.make_async_copy(
        ab_hbm.at[msrc1.at[pl.ds(0, BATCH)]], absbuf1, sem_ab1).wait()
    process(absbuf1, mdst1, msrc1, cnt_o)

    def regroup_body(r, _):
        rsp = jnp.full((16,), r, jnp.int32)
        idx = iota + r * 16
        for j in range(8):
            v = plsc.load_gather(accs[j], [idx])
            plsc.store_scatter(bhloc, [rsp, cols[j]], v)
        return 0
    lax.fori_loop(0, NLOC, regroup_body, 0)

    @pl.when(wid < NW - 1)
    def _():
        pltpu.sync_copy(bhloc, out_hbm.at[pl.ds(lo, NLOC)])

    @pl.when(wid == NW - 1)
    def _():
        pltpu.sync_copy(bhloc.at[pl.ds(0, last_rows)],
                        out_hbm.at[pl.ds((NW - 1) * NLOC, last_rows)])


def _sc_edge(ab, bh, src, dst):
    mesh = plsc.VectorSubcoreMesh(core_axis_name="c", subcore_axis_name="s")
    return pl.kernel(
        _sc_edge_body,
        out_type=jax.ShapeDtypeStruct((N_NODES, D), jnp.float32),
        mesh=mesh,
        compiler_params=pltpu.CompilerParams(needs_layout_passes=False),
        scratch_types=[
            *[pltpu.VMEM((NLOC * 16,), jnp.float32) for _ in range(8)],
            pltpu.VMEM((NLOC, D), jnp.float32),
            pltpu.VMEM((CE,), jnp.int32),
            pltpu.VMEM((CE,), jnp.int32),
            pltpu.VMEM((CE,), jnp.int32),
            pltpu.VMEM((CE,), jnp.int32),
            pltpu.VMEM((CE + BATCH,), jnp.int32),
            pltpu.VMEM((CE + BATCH,), jnp.int32),
            pltpu.VMEM((CE + BATCH,), jnp.int32),
            pltpu.VMEM((CE + BATCH,), jnp.int32),
            pltpu.VMEM((BATCH, 2 * D), jnp.float32),
            pltpu.VMEM((BATCH, 2 * D), jnp.float32),
            pltpu.SemaphoreType.DMA,
            pltpu.SemaphoreType.DMA,
            pltpu.SemaphoreType.DMA,
            pltpu.SemaphoreType.DMA,
        ],
    )(ab, bh, src, dst)


def _norm_body(h_ref, c_ref, norm_ref, out_ref):
    h = h_ref[...]
    c = c_ref[...]
    c = jnp.where(jnp.isfinite(c), c, 0.0)
    n2 = jnp.sum(h * h, axis=1, keepdims=True) + \
        jnp.sum(c * c, axis=1, keepdims=True)
    denom = jnp.maximum(jnp.sqrt(n2), 1e-12)
    scale = norm_ref[...] / denom
    out_ref[:, :D] = h * scale
    out_ref[:, D:] = c * scale


def _tc_normalize(h, c, norm):
    grid = (N_NODES // _RB,)
    return pl.pallas_call(
        _norm_body,
        grid=grid,
        in_specs=[
            pl.BlockSpec((_RB, D), lambda i: (i, 0)),
            pl.BlockSpec((_RB, D), lambda i: (i, 0)),
            pl.BlockSpec((_RB, 1), lambda i: (i, 0)),
        ],
        out_specs=pl.BlockSpec((_RB, 2 * D), lambda i: (i, 0)),
        out_shape=jax.ShapeDtypeStruct((N_NODES, 2 * D), jnp.float32),
    )(h, c, norm)


def kernel(x, edge_index, norm, W_A, b_A, W_B, b_B):
    src = edge_index[0].astype(jnp.int32)
    dst = edge_index[1].astype(jnp.int32)
    h, ab, bh = _tc_matmuls(x, norm, W_A, b_A, W_B, b_B)
    c = _sc_edge(ab, bh, src, dst)
    return _tc_normalize(h, c, norm)


# speedup vs baseline: 3.7297x; 1.0452x over previous
import jax
import jax.numpy as jnp
from jax import lax
from jax.experimental import pallas as pl
from jax.experimental.pallas import tpu as pltpu
from jax.experimental.pallas import tpu_sc as plsc

N_NODES = 10000
N_EDGES = 320000
D = 128

NW = 32
NLOC = 320
CE = 1600
NCHUNK = N_EDGES // CE
BATCH = 64
UNROLL = 2

NEG_INF = float("-inf")


_RB = 1000


def _mm_body(x_ref, norm_ref, wa_ref, ba_ref, wb_ref, bb_ref,
             h_ref, ab_ref, bh_ref):
    h = x_ref[...] * norm_ref[...]
    h_ref[...] = h
    ah = jnp.dot(h, wa_ref[...], preferred_element_type=jnp.float32) \
        + ba_ref[...]
    bh = jnp.dot(h, wb_ref[...], preferred_element_type=jnp.float32) \
        + bb_ref[...]
    ab_ref[:, :D] = ah
    ab_ref[:, D:] = bh
    bh_ref[...] = bh


def _tc_matmuls(x, norm, W_A, b_A, W_B, b_B):
    grid = (N_NODES // _RB,)
    return pl.pallas_call(
        _mm_body,
        grid=grid,
        in_specs=[
            pl.BlockSpec((_RB, D), lambda i: (i, 0)),
            pl.BlockSpec((_RB, 1), lambda i: (i, 0)),
            pl.BlockSpec((D, D), lambda i: (0, 0)),
            pl.BlockSpec((1, D), lambda i: (0, 0)),
            pl.BlockSpec((D, D), lambda i: (0, 0)),
            pl.BlockSpec((1, D), lambda i: (0, 0)),
        ],
        out_specs=[
            pl.BlockSpec((_RB, D), lambda i: (i, 0)),
            pl.BlockSpec((_RB, 2 * D), lambda i: (i, 0)),
            pl.BlockSpec((_RB, D), lambda i: (i, 0)),
        ],
        out_shape=[
            jax.ShapeDtypeStruct((N_NODES, D), jnp.float32),
            jax.ShapeDtypeStruct((N_NODES, 2 * D), jnp.float32),
            jax.ShapeDtypeStruct((N_NODES, D), jnp.float32),
        ],
    )(x, norm, W_A, b_A.reshape(1, D), W_B, b_B.reshape(1, D))


_EB = N_EDGES


def _pack_body(src_ref, dst_ref, pk_ref):
    pk_ref[...] = (dst_ref[...] << 16) | src_ref[...]


def _tc_pack(src, dst):
    grid = (N_EDGES // _EB,)
    return pl.pallas_call(
        _pack_body,
        grid=grid,
        in_specs=[
            pl.BlockSpec((_EB,), lambda i: (i,)),
            pl.BlockSpec((_EB,), lambda i: (i,)),
        ],
        out_specs=pl.BlockSpec((_EB,), lambda i: (i,)),
        out_shape=jax.ShapeDtypeStruct((N_EDGES,), jnp.int32),
    )(src, dst)


def _sc_edge_body(ab_hbm, bh_hbm, pk_hbm, out_hbm,
                  acc0, acc1, acc2, acc3, acc4, acc5, acc6, acc7,
                  bhloc, pkc0, pkc1,
                  mpk0, msrc0, mpk1, msrc1, absbuf0, absbuf1,
                  sem_stage, sem_ab0, sem_ab1, sem_slow):
    accs = (acc0, acc1, acc2, acc3, acc4, acc5, acc6, acc7)
    wid = lax.axis_index("s") * 2 + lax.axis_index("c")
    lo = wid * NLOC
    iota = lax.iota(jnp.int32, 16)
    cols = [iota + 16 * j for j in range(8)]
    lo_splat = jnp.full((16,), lo, jnp.int32)
    ninf = jnp.full((16,), NEG_INF, jnp.float32)
    last_rows = N_NODES - (NW - 1) * NLOC

    @pl.when(wid < NW - 1)
    def _():
        pltpu.sync_copy(bh_hbm.at[pl.ds(lo, NLOC)], bhloc)

    @pl.when(wid == NW - 1)
    def _():
        pltpu.sync_copy(bh_hbm.at[pl.ds((NW - 1) * NLOC, last_rows)],
                        bhloc.at[pl.ds(0, last_rows)])

    def init_body(r, _):
        idx = iota + r * 16
        for j in range(8):
            plsc.store_scatter(accs[j], [idx], ninf)
        return 0
    lax.fori_loop(0, NLOC, init_body, 0)

    pk_pad = jnp.full((16,), (jnp.int32(0) << 16) | 0, jnp.int32)

    def scan_chunk(ch, pkcr, mpk_r, msrc_r):
        pad_pk = (lo_splat << 16) | lo_splat

        def scan_body(i, off_v):
            pk = plsc.load_gather(pkcr, [iota + i * 16])
            rel = (pk >> 16) - lo_splat
            m = (rel >= 0) & (rel < NLOC)
            _, vv = plsc.sort_key_val(m.astype(jnp.int32), pk,
                                      descending=True)
            pos = off_v + iota
            plsc.store_scatter(mpk_r, [pos], vv)
            plsc.store_scatter(msrc_r, [pos], vv & 0xFFFF)
            return off_v + plsc.all_reduce_population_count(m)

        off_v = lax.fori_loop(0, CE // 16, scan_body,
                              jnp.zeros((16,), jnp.int32))
        for k in range(BATCH // 16):
            pidx = off_v + iota + 16 * k
            plsc.store_scatter(mpk_r, [pidx], pad_pk)
            plsc.store_scatter(msrc_r, [pidx], lo_splat)
        return jnp.max(off_v)

    def stage_issue(ch, pkcr):
        pltpu.async_copy(pk_hbm.at[pl.ds(ch * CE, CE)], pkcr, sem_stage)

    def stage_wait(pkcr):
        pltpu.make_async_copy(pk_hbm.at[pl.ds(0, CE)], pkcr,
                              sem_stage).wait()

    def process(absref, mpk_r, msrc_r, cnt):
        cnt_sp = jnp.full((16,), cnt, jnp.int32)
        nb = (cnt + BATCH - 1) // BATCH

        def batch_body(b, _):
            base = b * BATCH

            @pl.when(b > 0)
            def _():
                pltpu.async_copy(
                    ab_hbm.at[msrc_r.at[pl.ds(base, BATCH)]], absref,
                    sem_slow).wait()

            def quad_body(it, _):
                edges = []
                for u in range(UNROLL):
                    el = it * UNROLL + u
                    esp = jnp.full((16,), base + el, jnp.int32)
                    lsp = jnp.full((16,), el, jnp.int32)
                    rel = (plsc.load_gather(mpk_r, [esp]) >> 16) - lo_splat
                    aidx = (rel << 4) + iota
                    valid = esp < cnt_sp
                    edges.append((lsp, rel, aidx, valid))
                J = range(8)
                bb = [[plsc.load_gather(absref, [e[0], cols[j] + 128])
                       for j in J] for e in edges]
                bd = [[plsc.load_gather(bhloc, [e[1], cols[j]])
                       for j in J] for e in edges]
                t = [[bb[u][j] + bd[u][j] for j in J] for u in range(UNROLL)]
                ex = [[jnp.exp(-t[u][j]) for j in J] for u in range(UNROLL)]
                dn = [[1.0 + ex[u][j] for j in J] for u in range(UNROLL)]
                iv = [[1.0 / dn[u][j] for j in J] for u in range(UNROLL)]
                av = [[plsc.load_gather(absref, [e[0], cols[j]])
                       for j in J] for e in edges]
                for u, e in enumerate(edges):
                    for j in J:
                        msg = av[u][j] * iv[u][j]
                        cur = plsc.load_gather(accs[j], [e[2]])
                        plsc.store_scatter(accs[j], [e[2]],
                                           jnp.maximum(cur, msg), mask=e[3])
                return 0

            nq = jnp.minimum((cnt - base + UNROLL - 1) >> 1, BATCH // UNROLL)
            lax.fori_loop(0, nq, quad_body, 0)
            return 0

        lax.fori_loop(0, nb, batch_body, 0)

    stage_issue(0, pkc0)

    def pair_body(k, cnt_o):
        ch = 2 * k
        stage_wait(pkc0)
        stage_issue(ch + 1, pkc1)
        cnt_e = scan_chunk(ch, pkc0, mpk0, msrc0)
        d_ab0 = pltpu.async_copy(
            ab_hbm.at[msrc0.at[pl.ds(0, BATCH)]], absbuf0, sem_ab0)

        @pl.when(k > 0)
        def _():
            pltpu.make_async_copy(
                ab_hbm.at[msrc1.at[pl.ds(0, BATCH)]], absbuf1, sem_ab1).wait()
            process(absbuf1, mpk1, msrc1, cnt_o)

        stage_wait(pkc1)

        @pl.when(k < NCHUNK // 2 - 1)
        def _():
            stage_issue(ch + 2, pkc0)
        cnt_o_new = scan_chunk(ch + 1, pkc1, mpk1, msrc1)
        pltpu.async_copy(
            ab_hbm.at[msrc1.at[pl.ds(0, BATCH)]], absbuf1, sem_ab1)

        d_ab0.wait()
        process(absbuf0, mpk0, msrc0, cnt_e)
        return cnt_o_new

    cnt_o = lax.fori_loop(0, NCHUNK // 2, pair_body, jnp.int32(0))
    pltpu.make_async_copy(
        ab_hbm.at[msrc1.at[pl.ds(0, BATCH)]], absbuf1, sem_ab1).wait()
    process(absbuf1, mpk1, msrc1, cnt_o)

    def regroup_body(r, _):
        rsp = jnp.full((16,), r, jnp.int32)
        idx = iota + r * 16
        for j in range(8):
            v = plsc.load_gather(accs[j], [idx])
            plsc.store_scatter(bhloc, [rsp, cols[j]], v)
        return 0
    lax.fori_loop(0, NLOC, regroup_body, 0)

    @pl.when(wid < NW - 1)
    def _():
        pltpu.sync_copy(bhloc, out_hbm.at[pl.ds(lo, NLOC)])

    @pl.when(wid == NW - 1)
    def _():
        pltpu.sync_copy(bhloc.at[pl.ds(0, last_rows)],
                        out_hbm.at[pl.ds((NW - 1) * NLOC, last_rows)])


def _sc_edge(ab, bh, pk):
    mesh = plsc.VectorSubcoreMesh(core_axis_name="c", subcore_axis_name="s")
    return pl.kernel(
        _sc_edge_body,
        out_type=jax.ShapeDtypeStruct((N_NODES, D), jnp.float32),
        mesh=mesh,
        compiler_params=pltpu.CompilerParams(needs_layout_passes=False),
        scratch_types=[
            *[pltpu.VMEM((NLOC * 16,), jnp.float32) for _ in range(8)],
            pltpu.VMEM((NLOC, D), jnp.float32),
            pltpu.VMEM((CE,), jnp.int32),
            pltpu.VMEM((CE,), jnp.int32),
            pltpu.VMEM((CE + BATCH,), jnp.int32),
            pltpu.VMEM((CE + BATCH,), jnp.int32),
            pltpu.VMEM((CE + BATCH,), jnp.int32),
            pltpu.VMEM((CE + BATCH,), jnp.int32),
            pltpu.VMEM((BATCH, 2 * D), jnp.float32),
            pltpu.VMEM((BATCH, 2 * D), jnp.float32),
            pltpu.SemaphoreType.DMA,
            pltpu.SemaphoreType.DMA,
            pltpu.SemaphoreType.DMA,
            pltpu.SemaphoreType.DMA,
        ],
    )(ab, bh, pk)


def _norm_body(h_ref, c_ref, norm_ref, out_ref):
    h = h_ref[...]
    c = c_ref[...]
    c = jnp.where(jnp.isfinite(c), c, 0.0)
    n2 = jnp.sum(h * h, axis=1, keepdims=True) + \
        jnp.sum(c * c, axis=1, keepdims=True)
    denom = jnp.maximum(jnp.sqrt(n2), 1e-12)
    scale = norm_ref[...] / denom
    out_ref[:, :D] = h * scale
    out_ref[:, D:] = c * scale


def _tc_normalize(h, c, norm):
    grid = (N_NODES // _RB,)
    return pl.pallas_call(
        _norm_body,
        grid=grid,
        in_specs=[
            pl.BlockSpec((_RB, D), lambda i: (i, 0)),
            pl.BlockSpec((_RB, D), lambda i: (i, 0)),
            pl.BlockSpec((_RB, 1), lambda i: (i, 0)),
        ],
        out_specs=pl.BlockSpec((_RB, 2 * D), lambda i: (i, 0)),
        out_shape=jax.ShapeDtypeStruct((N_NODES, 2 * D), jnp.float32),
    )(h, c, norm)


def kernel(x, edge_index, norm, W_A, b_A, W_B, b_B):
    ei = edge_index.astype(jnp.int32)
    pk = _tc_pack(ei[0], ei[1])
    h, ab, bh = _tc_matmuls(x, norm, W_A, b_A, W_B, b_B)
    c = _sc_edge(ab, bh, pk)
    return _tc_normalize(h, c, norm)


# speedup vs baseline: 4.1838x; 1.1218x over previous
import jax
import jax.numpy as jnp
from jax import lax
from jax.experimental import pallas as pl
from jax.experimental.pallas import tpu as pltpu
from jax.experimental.pallas import tpu_sc as plsc

N_NODES = 10000
N_EDGES = 320000
D = 128

NW = 32
NLOC = 320
CE = 1600
NCHUNK = N_EDGES // CE
BATCH = 64
UNROLL = 4
WAVE = 4

NEG_INF = float("-inf")


_RB = 1000


def _mm_body(x_ref, norm_ref, wa_ref, ba_ref, wb_ref, bb_ref,
             h_ref, ab_ref, bh_ref):
    h = x_ref[...] * norm_ref[...]
    h_ref[...] = h
    ah = jnp.dot(h, wa_ref[...], preferred_element_type=jnp.float32) \
        + ba_ref[...]
    bh = jnp.dot(h, wb_ref[...], preferred_element_type=jnp.float32) \
        + bb_ref[...]
    ab_ref[:, :D] = ah
    ab_ref[:, D:] = bh
    bh_ref[...] = bh


def _tc_matmuls(x, norm, W_A, b_A, W_B, b_B):
    grid = (N_NODES // _RB,)
    return pl.pallas_call(
        _mm_body,
        grid=grid,
        in_specs=[
            pl.BlockSpec((_RB, D), lambda i: (i, 0)),
            pl.BlockSpec((_RB, 1), lambda i: (i, 0)),
            pl.BlockSpec((D, D), lambda i: (0, 0)),
            pl.BlockSpec((1, D), lambda i: (0, 0)),
            pl.BlockSpec((D, D), lambda i: (0, 0)),
            pl.BlockSpec((1, D), lambda i: (0, 0)),
        ],
        out_specs=[
            pl.BlockSpec((_RB, D), lambda i: (i, 0)),
            pl.BlockSpec((_RB, 2 * D), lambda i: (i, 0)),
            pl.BlockSpec((_RB, D), lambda i: (i, 0)),
        ],
        out_shape=[
            jax.ShapeDtypeStruct((N_NODES, D), jnp.float32),
            jax.ShapeDtypeStruct((N_NODES, 2 * D), jnp.float32),
            jax.ShapeDtypeStruct((N_NODES, D), jnp.float32),
        ],
    )(x, norm, W_A, b_A.reshape(1, D), W_B, b_B.reshape(1, D))


_EB = N_EDGES


def _pack_body(src_ref, dst_ref, pk_ref):
    pk_ref[...] = (dst_ref[...] << 16) | src_ref[...]


def _tc_pack(src, dst):
    grid = (N_EDGES // _EB,)
    return pl.pallas_call(
        _pack_body,
        grid=grid,
        in_specs=[
            pl.BlockSpec((_EB,), lambda i: (i,)),
            pl.BlockSpec((_EB,), lambda i: (i,)),
        ],
        out_specs=pl.BlockSpec((_EB,), lambda i: (i,)),
        out_shape=jax.ShapeDtypeStruct((N_EDGES,), jnp.int32),
    )(src, dst)


def _sc_edge_body(ab_hbm, bh_hbm, pk_hbm, out_hbm,
                  acc0, acc1, acc2, acc3, acc4, acc5, acc6, acc7,
                  bhloc, pkc0, pkc1,
                  mpk0, msrc0, mpk1, msrc1, absbuf0, absbuf1,
                  sem_stage, sem_ab0, sem_ab1, sem_slow):
    accs = (acc0, acc1, acc2, acc3, acc4, acc5, acc6, acc7)
    wid = lax.axis_index("s") * 2 + lax.axis_index("c")
    lo = wid * NLOC
    iota = lax.iota(jnp.int32, 16)
    cols = [iota + 16 * j for j in range(8)]
    lo_splat = jnp.full((16,), lo, jnp.int32)
    ninf = jnp.full((16,), NEG_INF, jnp.float32)
    last_rows = N_NODES - (NW - 1) * NLOC

    @pl.when(wid < NW - 1)
    def _():
        pltpu.sync_copy(bh_hbm.at[pl.ds(lo, NLOC)], bhloc)

    @pl.when(wid == NW - 1)
    def _():
        pltpu.sync_copy(bh_hbm.at[pl.ds((NW - 1) * NLOC, last_rows)],
                        bhloc.at[pl.ds(0, last_rows)])

    def init_body(r, _):
        idx = iota + r * 16
        for j in range(8):
            plsc.store_scatter(accs[j], [idx], ninf)
        return 0
    lax.fori_loop(0, NLOC, init_body, 0)

    pk_pad = jnp.full((16,), (jnp.int32(0) << 16) | 0, jnp.int32)

    def scan_chunk(ch, pkcr, mpk_r, msrc_r):
        pad_pk = (lo_splat << 16) | lo_splat

        def scan_body(i, off_v):
            pk0 = plsc.load_gather(pkcr, [iota + i * 32])
            pk1 = plsc.load_gather(pkcr, [iota + i * 32 + 16])
            m0 = ((pk0 >> 16) - lo_splat).astype(jnp.uint32) < NLOC
            m1 = ((pk1 >> 16) - lo_splat).astype(jnp.uint32) < NLOC
            _, vv0 = plsc.sort_key_val(m0.astype(jnp.int32), pk0,
                                       descending=True)
            _, vv1 = plsc.sort_key_val(m1.astype(jnp.int32), pk1,
                                       descending=True)
            c0 = plsc.all_reduce_population_count(m0)
            plsc.store_scatter(mpk_r, [off_v + iota], vv0)
            plsc.store_scatter(msrc_r, [off_v + iota], vv0 & 0xFFFF)
            off_v1 = off_v + c0
            plsc.store_scatter(mpk_r, [off_v1 + iota], vv1)
            plsc.store_scatter(msrc_r, [off_v1 + iota], vv1 & 0xFFFF)
            return off_v1 + plsc.all_reduce_population_count(m1)

        off_v = lax.fori_loop(0, CE // 32, scan_body,
                              jnp.zeros((16,), jnp.int32))
        for k in range(BATCH // 16):
            pidx = off_v + iota + 16 * k
            plsc.store_scatter(mpk_r, [pidx], pad_pk)
            plsc.store_scatter(msrc_r, [pidx], lo_splat)
        return jnp.max(off_v)

    def stage_issue(ch, pkcr):
        pltpu.async_copy(pk_hbm.at[pl.ds(ch * CE, CE)], pkcr, sem_stage)

    def stage_wait(pkcr):
        pltpu.make_async_copy(pk_hbm.at[pl.ds(0, CE)], pkcr,
                              sem_stage).wait()

    def process(absref, mpk_r, msrc_r, cnt):
        cnt_sp = jnp.full((16,), cnt, jnp.int32)
        nb = (cnt + BATCH - 1) // BATCH

        def batch_body(b, _):
            base = b * BATCH

            @pl.when(b > 0)
            def _():
                pltpu.async_copy(
                    ab_hbm.at[msrc_r.at[pl.ds(base, BATCH)]], absref,
                    sem_slow).wait()

            def quad_body(it, _):
                edges = []
                for u in range(UNROLL):
                    el = it * UNROLL + u
                    esp = jnp.full((16,), base + el, jnp.int32)
                    lsp = jnp.full((16,), el, jnp.int32)
                    rel = (plsc.load_gather(mpk_r, [esp]) >> 16) - lo_splat
                    aidx = (rel << 4) + iota
                    valid = esp < cnt_sp
                    edges.append((lsp, rel, aidx, valid))
                for j0 in range(0, 8, WAVE):
                    J = range(j0, j0 + WAVE)
                    bb = [[plsc.load_gather(absref, [e[0], cols[j] + 128])
                           for j in J] for e in edges]
                    bd = [[plsc.load_gather(bhloc, [e[1], cols[j]])
                           for j in J] for e in edges]
                    t = [[bb[u][i] + bd[u][i] for i in range(WAVE)]
                         for u in range(UNROLL)]
                    ex = [[jnp.exp(-t[u][i]) for i in range(WAVE)]
                          for u in range(UNROLL)]
                    dn = [[1.0 + ex[u][i] for i in range(WAVE)]
                          for u in range(UNROLL)]
                    iv = [[1.0 / dn[u][i] for i in range(WAVE)]
                          for u in range(UNROLL)]
                    av = [[plsc.load_gather(absref, [e[0], cols[j]])
                           for j in J] for e in edges]
                    for u, e in enumerate(edges):
                        for i, j in enumerate(J):
                            msg = av[u][i] * iv[u][i]
                            cur = plsc.load_gather(accs[j], [e[2]])
                            plsc.store_scatter(accs[j], [e[2]],
                                               jnp.maximum(cur, msg),
                                               mask=e[3])
                return 0

            nq = jnp.minimum((cnt - base + UNROLL - 1) >> 2, BATCH // UNROLL)
            lax.fori_loop(0, nq, quad_body, 0)
            return 0

        lax.fori_loop(0, nb, batch_body, 0)

    stage_issue(0, pkc0)

    def pair_body(k, cnt_o):
        ch = 2 * k
        stage_wait(pkc0)
        stage_issue(ch + 1, pkc1)
        cnt_e = scan_chunk(ch, pkc0, mpk0, msrc0)
        d_ab0 = pltpu.async_copy(
            ab_hbm.at[msrc0.at[pl.ds(0, BATCH)]], absbuf0, sem_ab0)

        @pl.when(k > 0)
        def _():
            pltpu.make_async_copy(
                ab_hbm.at[msrc1.at[pl.ds(0, BATCH)]], absbuf1, sem_ab1).wait()
            process(absbuf1, mpk1, msrc1, cnt_o)

        stage_wait(pkc1)

        @pl.when(k < NCHUNK // 2 - 1)
        def _():
            stage_issue(ch + 2, pkc0)
        cnt_o_new = scan_chunk(ch + 1, pkc1, mpk1, msrc1)
        pltpu.async_copy(
            ab_hbm.at[msrc1.at[pl.ds(0, BATCH)]], absbuf1, sem_ab1)

        d_ab0.wait()
        process(absbuf0, mpk0, msrc0, cnt_e)
        return cnt_o_new

    cnt_o = lax.fori_loop(0, NCHUNK // 2, pair_body, jnp.int32(0))
    pltpu.make_async_copy(
        ab_hbm.at[msrc1.at[pl.ds(0, BATCH)]], absbuf1, sem_ab1).wait()
    process(absbuf1, mpk1, msrc1, cnt_o)

    def regroup_body(r, _):
        rsp = jnp.full((16,), r, jnp.int32)
        idx = iota + r * 16
        for j in range(8):
            v = plsc.load_gather(accs[j], [idx])
            plsc.store_scatter(bhloc, [rsp, cols[j]], v)
        return 0
    lax.fori_loop(0, NLOC, regroup_body, 0)

    @pl.when(wid < NW - 1)
    def _():
        pltpu.sync_copy(bhloc, out_hbm.at[pl.ds(lo, NLOC)])

    @pl.when(wid == NW - 1)
    def _():
        pltpu.sync_copy(bhloc.at[pl.ds(0, last_rows)],
                        out_hbm.at[pl.ds((NW - 1) * NLOC, last_rows)])


def _sc_edge(ab, bh, pk):
    mesh = plsc.VectorSubcoreMesh(core_axis_name="c", subcore_axis_name="s")
    return pl.kernel(
        _sc_edge_body,
        out_type=jax.ShapeDtypeStruct((N_NODES, D), jnp.float32),
        mesh=mesh,
        compiler_params=pltpu.CompilerParams(needs_layout_passes=False),
        scratch_types=[
            *[pltpu.VMEM((NLOC * 16,), jnp.float32) for _ in range(8)],
            pltpu.VMEM((NLOC, D), jnp.float32),
            pltpu.VMEM((CE,), jnp.int32),
            pltpu.VMEM((CE,), jnp.int32),
            pltpu.VMEM((CE + BATCH,), jnp.int32),
            pltpu.VMEM((CE + BATCH,), jnp.int32),
            pltpu.VMEM((CE + BATCH,), jnp.int32),
            pltpu.VMEM((CE + BATCH,), jnp.int32),
            pltpu.VMEM((BATCH, 2 * D), jnp.float32),
            pltpu.VMEM((BATCH, 2 * D), jnp.float32),
            pltpu.SemaphoreType.DMA,
            pltpu.SemaphoreType.DMA,
            pltpu.SemaphoreType.DMA,
            pltpu.SemaphoreType.DMA,
        ],
    )(ab, bh, pk)


def _norm_body(h_ref, c_ref, norm_ref, out_ref):
    h = h_ref[...]
    c = c_ref[...]
    c = jnp.where(jnp.isfinite(c), c, 0.0)
    n2 = jnp.sum(h * h, axis=1, keepdims=True) + \
        jnp.sum(c * c, axis=1, keepdims=True)
    denom = jnp.maximum(jnp.sqrt(n2), 1e-12)
    scale = norm_ref[...] / denom
    out_ref[:, :D] = h * scale
    out_ref[:, D:] = c * scale


def _tc_normalize(h, c, norm):
    grid = (N_NODES // _RB,)
    return pl.pallas_call(
        _norm_body,
        grid=grid,
        in_specs=[
            pl.BlockSpec((_RB, D), lambda i: (i, 0)),
            pl.BlockSpec((_RB, D), lambda i: (i, 0)),
            pl.BlockSpec((_RB, 1), lambda i: (i, 0)),
        ],
        out_specs=pl.BlockSpec((_RB, 2 * D), lambda i: (i, 0)),
        out_shape=jax.ShapeDtypeStruct((N_NODES, 2 * D), jnp.float32),
    )(h, c, norm)


def kernel(x, edge_index, norm, W_A, b_A, W_B, b_B):
    ei = edge_index.astype(jnp.int32)
    pk = _tc_pack(ei[0], ei[1])
    h, ab, bh = _tc_matmuls(x, norm, W_A, b_A, W_B, b_B)
    c = _sc_edge(ab, bh, pk)
    return _tc_normalize(h, c, norm)
